# Initial kernel scaffold; baseline (speedup 1.0000x reference)
import jax
import jax.numpy as jnp
from jax.experimental import pallas as pl


def kernel(current_map, point_cloud, weights):
    raise NotImplementedError("write your pallas kernel here")


# speedup vs baseline: 15.7647x; 15.7647x over previous
import functools

import jax
import jax.numpy as jnp
import numpy as np
from jax import lax
from jax.experimental import pallas as pl
from jax.experimental.pallas import tpu as pltpu
from jax.experimental.pallas import tpu_sc as plsc

_GRID = (256, 256, 32)
_NUM_CLASSES = 21
_MIN_B = np.array([-25.6, -25.6, -2.0], np.float32)
_MAX_B = np.array([25.6, 25.6, 4.4], np.float32)

_NC = 2
_NS = 16
_NW = _NC * _NS
_LANES = 16


def _sc_scatter_add(ref, idx, x, mask):
  plsc.addupdate_scatter(ref, [idx], x, mask=mask)


def _sc_append_compact(ref, x, mask):
  sk, _sv, _om = plsc.sort_key_val(x, x, mask=mask)
  ref[...] = sk


def _sc_load_gather(ref, idx):
  return plsc.load_gather(ref, [idx])


def _sc_scan_count(x, mask):
  return plsc.scan_count(x, mask=mask)


def _axis_index(name):
  return lax.axis_index(name)


def _sync_copy(src, dst):
  pltpu.sync_copy(src, dst)


def _make_body(X, Y, Z, C, n_pad, chunk, clcap, slcap):
  assert X % _NW == 0 and Y % 2 == 0 and n_pad % chunk == 0
  assert chunk % _LANES == 0
  xpw = X // _NW
  yh = Y // 2
  zc = Z * C
  reg = yh * zc
  n_chunks = n_pad // chunk
  vecs_per_chunk = chunk // _LANES

  minb = [float(v) for v in _MIN_B]
  maxb = [float(v) for v in _MAX_B]
  vs = (np.asarray(_MAX_B) - np.asarray(_MIN_B)) / np.asarray(
      (X, Y, Z), np.float32)
  inv_vs = [float(np.float32(1.0) / v) for v in vs]

  lane_iota = lambda: lax.iota(jnp.int32, _LANES)

  def body(map_hbm, px_hbm, py_hbm, pz_hbm, pc_hbm, w_hbm, out_hbm,
           acc, clist, slist, pbx, pby, pbz, pbc, filt):
    wid = _axis_index("s") * _NC + _axis_index("c")
    x_lo = wid * xpw
    win_lo = x_lo - 1

    _sync_copy(w_hbm, filt)
    v0 = filt[pl.ds(0, _LANES)]
    v0 = 1.0 / (1.0 + jnp.exp(-v0))
    v0 = jnp.where(lane_iota() == 13, 1.0, v0)
    filt[pl.ds(0, _LANES)] = v0
    v1 = filt[pl.ds(_LANES, _LANES)]
    v1 = 1.0 / (1.0 + jnp.exp(-v1))
    filt[pl.ds(_LANES, _LANES)] = v1

    def chunk_body(ci, n):
      base = ci * chunk
      _sync_copy(px_hbm.at[pl.ds(base, chunk)], pbx)
      _sync_copy(py_hbm.at[pl.ds(base, chunk)], pby)
      _sync_copy(pz_hbm.at[pl.ds(base, chunk)], pbz)
      _sync_copy(pc_hbm.at[pl.ds(base, chunk)], pbc)

      def vec_body(i, n):
        off = i * _LANES
        xv = pbx[pl.ds(off, _LANES)]
        yv = pby[pl.ds(off, _LANES)]
        zv = pbz[pl.ds(off, _LANES)]
        cv = pbc[pl.ds(off, _LANES)]
        fx = (xv - minb[0]) * inv_vs[0]
        fy = (yv - minb[1]) * inv_vs[1]
        fz = (zv - minb[2]) * inv_vs[2]
        ix = jnp.clip(fx.astype(jnp.int32), 0, X - 1)
        iy = jnp.clip(fy.astype(jnp.int32), 0, Y - 1)
        iz = jnp.clip(fz.astype(jnp.int32), 0, Z - 1)
        ic = jnp.clip(cv.astype(jnp.int32), 0, C - 1)
        valid = ((xv >= minb[0]) & (xv < maxb[0])
                 & (yv >= minb[1]) & (yv < maxb[1])
                 & (zv >= minb[2]) & (zv < maxb[2]))
        m = valid & (ix >= win_lo) & (ix <= x_lo + xpw)
        code = ((ix - win_lo) << 18) | (iy << 10) | (iz << 5) | ic
        noff = jnp.minimum(n, clcap - _LANES)
        _sc_append_compact(clist.at[pl.ds(noff, _LANES)], code, m)
        return n + jnp.sum(m.astype(jnp.int32))

      return lax.fori_loop(0, vecs_per_chunk, vec_body, n)

    n_pts = lax.fori_loop(0, n_chunks, chunk_body, jnp.int32(0))

    def slab_body(s, _):
      def filt_body(i, ns):
        off = i * _LANES
        codes = clist[pl.ds(off, _LANES)]
        lm = (lane_iota() + off) < n_pts
        ixl = codes >> 18
        m = lm & (ixl >= s) & (ixl <= s + 2)
        noff = jnp.minimum(ns, slcap - _LANES)
        _sc_append_compact(slist.at[pl.ds(noff, _LANES)], codes, m)
        return ns + jnp.sum(m.astype(jnp.int32))

      n_vecs = (n_pts + _LANES - 1) // _LANES
      ns_pts = lax.fori_loop(0, n_vecs, filt_body, jnp.int32(0))
      sx = x_lo + s

      def half_body(h, _):
        _sync_copy(map_hbm.at[sx, h], acc)
        ylo = h * yh

        def pt_body(i, _):
          off = i * _LANES
          codes = slist[pl.ds(off, _LANES)]
          lm = (lane_iota() + off) < ns_pts
          ixl = codes >> 18
          iy = (codes >> 10) & 0xFF
          iz = (codes >> 5) & 0x1F
          ic = codes & 0x1F
          ly = iy - ylo
          m0 = lm & (ly >= -1) & (ly <= yh)
          bidx = ly * zc + iz * C + ic
          cnts, _lastm = _sc_scan_count(bidx, m0)
          minc = jnp.min(jnp.where(m0, cnts, jnp.int32(2**30)))
          maxc = jnp.max(jnp.where(m0, cnts, jnp.int32(-2**30)))
          kx = (ixl - 1 - s) + 1
          taps = []
          for dy in (-1, 0, 1):
            for dz in (-1, 0, 1):
              lyt = ly + dy
              izt = iz + dz
              mt = m0 & (lyt >= 0) & (lyt < yh) & (izt >= 0) & (izt < Z)
              widx = kx * 9 + (1 - dy) * 3 + (1 - dz)
              wv = _sc_load_gather(filt, jnp.clip(widx, 0, 31))
              tidx = jnp.clip(lyt * zc + izt * C + ic, 0, reg - 1)
              taps.append((tidx, wv, mt))

          def round_body(r, _):
            mr = cnts == r
            for tidx, wv, mt in taps:
              _sc_scatter_add(acc, tidx, wv, mt & mr)
            return 0

          lax.fori_loop(minc, maxc + 1, round_body, 0)
          return 0

        ns_vecs = (ns_pts + _LANES - 1) // _LANES
        lax.fori_loop(0, ns_vecs, pt_body, 0)
        _sync_copy(acc, out_hbm.at[sx, h])
        return 0

      lax.fori_loop(0, 2, half_body, 0)
      return 0

    lax.fori_loop(0, xpw, slab_body, 0)

  return body


def _make_kernel(X, Y, Z, C, n_pad, chunk, clcap, slcap):
  body = _make_body(X, Y, Z, C, n_pad, chunk, clcap, slcap)
  reg = (Y // 2) * Z * C
  mesh = plsc.VectorSubcoreMesh(
      core_axis_name="c", subcore_axis_name="s", num_cores=_NC,
      num_subcores=_NS)
  return pl.kernel(
      body,
      out_type=jax.ShapeDtypeStruct((X, 2, reg), jnp.float32),
      mesh=mesh,
      scratch_types=[
          pltpu.VMEM((reg,), jnp.float32),
          pltpu.VMEM((clcap,), jnp.int32),
          pltpu.VMEM((slcap,), jnp.int32),
          pltpu.VMEM((chunk,), jnp.float32),
          pltpu.VMEM((chunk,), jnp.float32),
          pltpu.VMEM((chunk,), jnp.float32),
          pltpu.VMEM((chunk,), jnp.float32),
          pltpu.VMEM((32,), jnp.float32),
      ],
      compiler_params=pltpu.CompilerParams(needs_layout_passes=False),
  )


@jax.jit
def kernel(current_map, point_cloud, weights):
  X, Y, Z, C = current_map.shape
  n = point_cloud.shape[0]
  chunk = 2048
  n_pad = ((n + chunk - 1) // chunk) * chunk
  pts = jnp.concatenate(
      [point_cloud,
       jnp.full((n_pad - n, 4), 1e30, point_cloud.dtype)], axis=0)
  px, py, pz, pc = [pts[:, i] for i in range(4)]
  w_flat = jnp.concatenate(
      [weights.reshape(-1), jnp.zeros((32 - 27,), weights.dtype)])
  map3 = current_map.reshape(X, 2, (Y // 2) * Z * C)
  k = _make_kernel(X, Y, Z, C, n_pad, chunk, clcap=16384, slcap=8192)
  out = k(map3, px, py, pz, pc, w_flat)
  return out.reshape(X, Y, Z, C)
